# Initial kernel scaffold; baseline (speedup 1.0000x reference)
import jax
import jax.numpy as jnp
from jax.experimental import pallas as pl


def kernel(atoms_h, graph_batch, W1, b1, W2, b2):
    raise NotImplementedError("write your pallas kernel here")


# speedup vs baseline: 1.1287x; 1.1287x over previous
import functools

import jax
import jax.numpy as jnp
from jax import lax
from jax.experimental import pallas as pl
from jax.experimental.pallas import tpu as pltpu
from jax.experimental.pallas import tpu_sc as plsc

HIDDEN = 128
N_MOL = 1024
N_SUB = 16
CHUNK = 6400
N_PAD = N_SUB * CHUNK
BLK = 2048
ACC = 2048
COLS = ACC // N_SUB


def _mlp_body(x_ref, w1_ref, b1_ref, w2_ref, b2_ref, o_ref):
    x = x_ref[...]
    h = jnp.dot(x, w1_ref[...], preferred_element_type=jnp.float32)
    h = h + b1_ref[...]
    h = h * jax.nn.sigmoid(h)
    e = jnp.dot(h, w2_ref[...], preferred_element_type=jnp.float32)
    o_ref[...] = e + b2_ref[...]


def _atomic_energies(x_pad, W1, b1, W2, b2):
    grid = (N_PAD // BLK,)
    return pl.pallas_call(
        _mlp_body,
        grid=grid,
        in_specs=[
            pl.BlockSpec((BLK, HIDDEN), lambda i: (i, 0)),
            pl.BlockSpec((HIDDEN, HIDDEN), lambda i: (0, 0)),
            pl.BlockSpec((1, HIDDEN), lambda i: (0, 0)),
            pl.BlockSpec((HIDDEN, 1), lambda i: (0, 0)),
            pl.BlockSpec((1, 1), lambda i: (0, 0)),
        ],
        out_specs=pl.BlockSpec((BLK, 1), lambda i: (i, 0)),
        out_shape=jax.ShapeDtypeStruct((N_PAD, 1), jnp.float32),
        compiler_params=pltpu.CompilerParams(
            dimension_semantics=("arbitrary",),
        ),
    )(x_pad, W1, b1.reshape(1, HIDDEN), W2, b2.reshape(1, 1))


def _seg_body(e_hbm, id_hbm, out_hbm, e_v, id_v, acc_v, shr, tmp_v, res_v):
    w = lax.axis_index("s")
    base = w * CHUNK

    pltpu.sync_copy(e_hbm.at[pl.ds(base, CHUNK)], e_v)
    pltpu.sync_copy(id_hbm.at[pl.ds(base, CHUNK)], id_v)

    zeros = jnp.zeros((16,), jnp.float32)

    def zero_body(j, carry):
        acc_v[pl.ds(j * 16, 16)] = zeros
        return carry

    lax.fori_loop(0, ACC // 16, zero_body, 0, unroll=8)

    def scat_body(j, carry):
        idx = id_v[pl.ds(j * 16, 16)]
        val = e_v[pl.ds(j * 16, 16)]
        plsc.addupdate_scatter(acc_v, [idx], val)
        return carry

    lax.fori_loop(0, CHUNK // 16, scat_body, 0, unroll=8)

    pltpu.sync_copy(acc_v, shr.at[pl.ds(w * ACC, ACC)])
    plsc.subcore_barrier()
    for r in range(N_SUB):
        pltpu.sync_copy(
            shr.at[pl.ds(r * ACC + w * COLS, COLS)],
            tmp_v.at[pl.ds(r * COLS, COLS)],
        )

    for c in range(COLS // 16):
        s = jnp.zeros((16,), jnp.float32)
        for r in range(N_SUB):
            s = s + tmp_v[pl.ds(r * COLS + c * 16, 16)]
        res_v[pl.ds(c * 16, 16)] = s

    pltpu.sync_copy(res_v, out_hbm.at[pl.ds(w * COLS, COLS)])


def _segment_sum(energies, ids):
    mesh = plsc.VectorSubcoreMesh(
        core_axis_name="c", subcore_axis_name="s", num_cores=1
    )
    fn = pl.kernel(
        _seg_body,
        mesh=mesh,
        out_type=jax.ShapeDtypeStruct((ACC,), jnp.float32),
        scratch_types=[
            pltpu.VMEM((CHUNK,), jnp.float32),
            pltpu.VMEM((CHUNK,), jnp.int32),
            pltpu.VMEM((ACC,), jnp.float32),
            pltpu.VMEM_SHARED((N_SUB * ACC,), jnp.float32),
            pltpu.VMEM((N_SUB * COLS,), jnp.float32),
            pltpu.VMEM((COLS,), jnp.float32),
        ],
        compiler_params=pltpu.CompilerParams(needs_layout_passes=False),
    )
    return fn(energies, ids)


@jax.jit
def kernel(atoms_h, graph_batch, W1, b1, W2, b2):
    n = atoms_h.shape[0]
    x_pad = jnp.pad(atoms_h, ((0, N_PAD - n), (0, 0)))
    ids = jnp.pad(
        graph_batch.astype(jnp.int32), (0, N_PAD - n), constant_values=N_MOL
    )
    energies = _atomic_energies(x_pad, W1, b1, W2, b2).reshape(N_PAD)
    acc = _segment_sum(energies, ids)
    return acc[:N_MOL]


# speedup vs baseline: 1.4645x; 1.2975x over previous
import functools

import jax
import jax.numpy as jnp
from jax import lax
from jax.experimental import pallas as pl
from jax.experimental.pallas import tpu as pltpu
from jax.experimental.pallas import tpu_sc as plsc

HIDDEN = 128
N_MOL = 1024
N_SUB = 16
CHUNK = 6400
N_PAD = N_SUB * CHUNK
BLK = 2000
ACC = 2048
COLS = ACC // N_SUB


def _mlp_body(x_ref, w1_ref, b1_ref, w2_ref, b2_ref, o_ref):
    x = x_ref[...]
    h = jnp.dot(x, w1_ref[...], preferred_element_type=jnp.float32)
    h = h + b1_ref[...]
    h = h * jax.nn.sigmoid(h)
    e = jnp.dot(h, w2_ref[...], preferred_element_type=jnp.float32)
    o_ref[...] = e + b2_ref[...]


def _atomic_energies(x, W1, b1, W2, b2):
    n = x.shape[0]
    grid = (n // BLK,)
    return pl.pallas_call(
        _mlp_body,
        grid=grid,
        in_specs=[
            pl.BlockSpec((BLK, HIDDEN), lambda i: (i, 0)),
            pl.BlockSpec((HIDDEN, HIDDEN), lambda i: (0, 0)),
            pl.BlockSpec((1, HIDDEN), lambda i: (0, 0)),
            pl.BlockSpec((HIDDEN, 1), lambda i: (0, 0)),
            pl.BlockSpec((1, 1), lambda i: (0, 0)),
        ],
        out_specs=pl.BlockSpec((BLK, 1), lambda i: (i, 0)),
        out_shape=jax.ShapeDtypeStruct((n, 1), jnp.float32),
        compiler_params=pltpu.CompilerParams(
            dimension_semantics=("arbitrary",),
        ),
    )(x_pad, W1, b1.reshape(1, HIDDEN), W2, b2.reshape(1, 1))


def _seg_body(e_hbm, id_hbm, out_hbm, e_v, id_v, acc_v, shr, tmp_v, res_v):
    w = lax.axis_index("s")
    base = w * CHUNK

    pltpu.sync_copy(e_hbm.at[pl.ds(base, CHUNK)], e_v)
    pltpu.sync_copy(id_hbm.at[pl.ds(base, CHUNK)], id_v)

    zeros = jnp.zeros((16,), jnp.float32)

    def zero_body(j, carry):
        acc_v[pl.ds(j * 16, 16)] = zeros
        return carry

    lax.fori_loop(0, ACC // 16, zero_body, 0, unroll=8)

    def scat_body(j, carry):
        idx = id_v[pl.ds(j * 16, 16)]
        val = e_v[pl.ds(j * 16, 16)]
        plsc.addupdate_scatter(acc_v, [idx], val)
        return carry

    lax.fori_loop(0, CHUNK // 16, scat_body, 0, unroll=8)

    pltpu.sync_copy(acc_v, shr.at[pl.ds(w * ACC, ACC)])
    plsc.subcore_barrier()
    for r in range(N_SUB):
        pltpu.sync_copy(
            shr.at[pl.ds(r * ACC + w * COLS, COLS)],
            tmp_v.at[pl.ds(r * COLS, COLS)],
        )

    for c in range(COLS // 16):
        s = jnp.zeros((16,), jnp.float32)
        for r in range(N_SUB):
            s = s + tmp_v[pl.ds(r * COLS + c * 16, 16)]
        res_v[pl.ds(c * 16, 16)] = s

    pltpu.sync_copy(res_v, out_hbm.at[pl.ds(w * COLS, COLS)])


def _segment_sum(energies, ids):
    mesh = plsc.VectorSubcoreMesh(
        core_axis_name="c", subcore_axis_name="s", num_cores=1
    )
    fn = pl.kernel(
        _seg_body,
        mesh=mesh,
        out_type=jax.ShapeDtypeStruct((ACC,), jnp.float32),
        scratch_types=[
            pltpu.VMEM((CHUNK,), jnp.float32),
            pltpu.VMEM((CHUNK,), jnp.int32),
            pltpu.VMEM((ACC,), jnp.float32),
            pltpu.VMEM_SHARED((N_SUB * ACC,), jnp.float32),
            pltpu.VMEM((N_SUB * COLS,), jnp.float32),
            pltpu.VMEM((COLS,), jnp.float32),
        ],
        compiler_params=pltpu.CompilerParams(needs_layout_passes=False),
    )
    return fn(energies, ids)


@jax.jit
def kernel(atoms_h, graph_batch, W1, b1, W2, b2):
    n = atoms_h.shape[0]
    ids = jnp.pad(
        graph_batch.astype(jnp.int32), (0, N_PAD - n), constant_values=N_MOL
    )
    energies = _atomic_energies(atoms_h, W1, b1, W2, b2).reshape(n)
    energies = jnp.pad(energies, (0, N_PAD - n))
    acc = _segment_sum(energies, ids)
    return acc[:N_MOL]


# speedup vs baseline: 1.4842x; 1.0134x over previous
import functools

import jax
import jax.numpy as jnp
from jax import lax
from jax.experimental import pallas as pl
from jax.experimental.pallas import tpu as pltpu
from jax.experimental.pallas import tpu_sc as plsc

HIDDEN = 128
N_MOL = 1024
N_SUB = 16
CHUNK = 6400
N_PAD = N_SUB * CHUNK
BLK = 2000
ACC = 2048
COLS = ACC // N_SUB


def _mlp_body(x_ref, w1_ref, b1_ref, w2_ref, b2_ref, o_ref):
    x = x_ref[...]
    h = jnp.dot(x, w1_ref[...], preferred_element_type=jnp.float32)
    h = h + b1_ref[...]
    h = h * jax.nn.sigmoid(h)
    e = jnp.dot(h, w2_ref[...], preferred_element_type=jnp.float32)
    o_ref[...] = e + b2_ref[...]


def _atomic_energies(x, W1, b1, W2, b2):
    n = x.shape[0]
    grid = (n // BLK,)
    return pl.pallas_call(
        _mlp_body,
        grid=grid,
        in_specs=[
            pl.BlockSpec((BLK, HIDDEN), lambda i: (i, 0)),
            pl.BlockSpec((HIDDEN, HIDDEN), lambda i: (0, 0)),
            pl.BlockSpec((1, HIDDEN), lambda i: (0, 0)),
            pl.BlockSpec((HIDDEN, 1), lambda i: (0, 0)),
            pl.BlockSpec((1, 1), lambda i: (0, 0)),
        ],
        out_specs=pl.BlockSpec((BLK, 1), lambda i: (i, 0)),
        out_shape=jax.ShapeDtypeStruct((n, 1), jnp.float32),
        compiler_params=pltpu.CompilerParams(
            dimension_semantics=("arbitrary",),
        ),
    )(x, W1, b1.reshape(1, HIDDEN), W2, b2.reshape(1, 1))


def _seg_body(n, e_hbm, id_hbm, out_hbm, e_v, id_v, acc_v, shr, tmp_v, res_v):
    last = n - (N_SUB - 1) * CHUNK
    w = lax.axis_index("s")
    base = w * CHUNK

    @pl.when(w < N_SUB - 1)
    def _():
        pltpu.sync_copy(e_hbm.at[pl.ds(base, CHUNK)], e_v)
        pltpu.sync_copy(id_hbm.at[pl.ds(base, CHUNK)], id_v)

    @pl.when(w == N_SUB - 1)
    def _():
        pltpu.sync_copy(e_hbm.at[pl.ds(base, last)], e_v.at[pl.ds(0, last)])
        pltpu.sync_copy(id_hbm.at[pl.ds(base, last)], id_v.at[pl.ds(0, last)])

    zeros = jnp.zeros((16,), jnp.float32)

    def zero_body(j, carry):
        acc_v[pl.ds(j * 16, 16)] = zeros
        return carry

    lax.fori_loop(0, ACC // 16, zero_body, 0, unroll=8)

    lanes = lax.iota(jnp.int32, 16)

    def scat_body(j, carry):
        idx = id_v[pl.ds(j * 16, 16)]
        val = e_v[pl.ds(j * 16, 16)]
        valid = (base + j * 16 + lanes) < n
        plsc.addupdate_scatter(acc_v, [idx], val, mask=valid)
        return carry

    lax.fori_loop(0, CHUNK // 16, scat_body, 0, unroll=8)

    pltpu.sync_copy(acc_v, shr.at[pl.ds(w * ACC, ACC)])
    plsc.subcore_barrier()
    for r in range(N_SUB):
        pltpu.sync_copy(
            shr.at[pl.ds(r * ACC + w * COLS, COLS)],
            tmp_v.at[pl.ds(r * COLS, COLS)],
        )

    for c in range(COLS // 16):
        s = jnp.zeros((16,), jnp.float32)
        for r in range(N_SUB):
            s = s + tmp_v[pl.ds(r * COLS + c * 16, 16)]
        res_v[pl.ds(c * 16, 16)] = s

    pltpu.sync_copy(res_v, out_hbm.at[pl.ds(w * COLS, COLS)])


def _segment_sum(energies, ids):
    n = energies.shape[0]
    mesh = plsc.VectorSubcoreMesh(
        core_axis_name="c", subcore_axis_name="s", num_cores=1
    )
    fn = pl.kernel(
        functools.partial(_seg_body, n),
        mesh=mesh,
        out_type=jax.ShapeDtypeStruct((ACC,), jnp.float32),
        scratch_types=[
            pltpu.VMEM((CHUNK,), jnp.float32),
            pltpu.VMEM((CHUNK,), jnp.int32),
            pltpu.VMEM((ACC,), jnp.float32),
            pltpu.VMEM_SHARED((N_SUB * ACC,), jnp.float32),
            pltpu.VMEM((N_SUB * COLS,), jnp.float32),
            pltpu.VMEM((COLS,), jnp.float32),
        ],
        compiler_params=pltpu.CompilerParams(needs_layout_passes=False),
    )
    return fn(energies, ids)


@jax.jit
def kernel(atoms_h, graph_batch, W1, b1, W2, b2):
    n = atoms_h.shape[0]
    ids = graph_batch.astype(jnp.int32)
    energies = _atomic_energies(atoms_h, W1, b1, W2, b2).reshape(n)
    acc = _segment_sum(energies, ids)
    return acc[:N_MOL]


# speedup vs baseline: 1.9308x; 1.3009x over previous
import functools

import jax
import jax.numpy as jnp
from jax import lax
from jax.experimental import pallas as pl
from jax.experimental.pallas import tpu as pltpu
from jax.experimental.pallas import tpu_sc as plsc

HIDDEN = 128
N_MOL = 1024
N_SUB = 16
CHUNK = 6400
N_PAD = N_SUB * CHUNK
BLK = 2000
ACC = 2048
COLS = ACC // N_SUB


def _mlp_body(x_ref, w1_ref, b1_ref, w2_ref, b2_ref, o_ref):
    x = x_ref[...]
    h = jnp.dot(x, w1_ref[...], preferred_element_type=jnp.float32)
    h = h + b1_ref[...]
    h = h * jax.nn.sigmoid(h)
    e = jnp.dot(h, w2_ref[...], preferred_element_type=jnp.float32)
    o_ref[...] = (e + b2_ref[...]).reshape(1, BLK // 16, 16)


def _atomic_energies(x, W1, b1, W2, b2):
    n = x.shape[0]
    grid = (n // BLK,)
    return pl.pallas_call(
        _mlp_body,
        grid=grid,
        in_specs=[
            pl.BlockSpec((BLK, HIDDEN), lambda i: (i, 0)),
            pl.BlockSpec((HIDDEN, HIDDEN), lambda i: (0, 0)),
            pl.BlockSpec((1, HIDDEN), lambda i: (0, 0)),
            pl.BlockSpec((HIDDEN, 1), lambda i: (0, 0)),
            pl.BlockSpec((1, 1), lambda i: (0, 0)),
        ],
        out_specs=pl.BlockSpec((1, BLK // 16, 16), lambda i: (i, 0, 0)),
        out_shape=jax.ShapeDtypeStruct((n // BLK, BLK // 16, 16), jnp.float32),
        compiler_params=pltpu.CompilerParams(
            dimension_semantics=("arbitrary",),
        ),
    )(x, W1, b1.reshape(1, HIDDEN), W2, b2.reshape(1, 1))


def _seg_body(n, e_hbm, id_hbm, out_hbm, e_v, id_v, acc_v, shr, tmp_v, res_v):
    last = n - (N_SUB - 1) * CHUNK
    w = lax.axis_index("s")
    base = w * CHUNK

    @pl.when(w < N_SUB - 1)
    def _():
        pltpu.sync_copy(e_hbm.at[pl.ds(base, CHUNK)], e_v)
        pltpu.sync_copy(id_hbm.at[pl.ds(base, CHUNK)], id_v)

    @pl.when(w == N_SUB - 1)
    def _():
        pltpu.sync_copy(e_hbm.at[pl.ds(base, last)], e_v.at[pl.ds(0, last)])
        pltpu.sync_copy(id_hbm.at[pl.ds(base, last)], id_v.at[pl.ds(0, last)])

    zeros = jnp.zeros((16,), jnp.float32)

    def zero_body(j, carry):
        acc_v[pl.ds(j * 16, 16)] = zeros
        return carry

    lax.fori_loop(0, ACC // 16, zero_body, 0, unroll=8)

    lanes = lax.iota(jnp.int32, 16)

    def scat_body(j, carry):
        idx = id_v[pl.ds(j * 16, 16)]
        val = e_v[pl.ds(j * 16, 16)]
        valid = (base + j * 16 + lanes) < n
        plsc.addupdate_scatter(acc_v, [idx], val, mask=valid)
        return carry

    lax.fori_loop(0, CHUNK // 16, scat_body, 0, unroll=8)

    pltpu.sync_copy(acc_v, shr.at[pl.ds(w * ACC, ACC)])
    plsc.subcore_barrier()
    for r in range(N_SUB):
        pltpu.sync_copy(
            shr.at[pl.ds(r * ACC + w * COLS, COLS)],
            tmp_v.at[pl.ds(r * COLS, COLS)],
        )

    for c in range(COLS // 16):
        s = jnp.zeros((16,), jnp.float32)
        for r in range(N_SUB):
            s = s + tmp_v[pl.ds(r * COLS + c * 16, 16)]
        res_v[pl.ds(c * 16, 16)] = s

    pltpu.sync_copy(res_v, out_hbm.at[pl.ds(w * COLS, COLS)])


def _segment_sum(energies, ids):
    n = energies.shape[0]
    mesh = plsc.VectorSubcoreMesh(
        core_axis_name="c", subcore_axis_name="s", num_cores=1
    )
    fn = pl.kernel(
        functools.partial(_seg_body, n),
        mesh=mesh,
        out_type=jax.ShapeDtypeStruct((ACC,), jnp.float32),
        scratch_types=[
            pltpu.VMEM((CHUNK,), jnp.float32),
            pltpu.VMEM((CHUNK,), jnp.int32),
            pltpu.VMEM((ACC,), jnp.float32),
            pltpu.VMEM_SHARED((N_SUB * ACC,), jnp.float32),
            pltpu.VMEM((N_SUB * COLS,), jnp.float32),
            pltpu.VMEM((COLS,), jnp.float32),
        ],
        compiler_params=pltpu.CompilerParams(needs_layout_passes=False),
    )
    return fn(energies, ids)


@jax.jit
def kernel(atoms_h, graph_batch, W1, b1, W2, b2):
    n = atoms_h.shape[0]
    ids = graph_batch.astype(jnp.int32)
    energies = _atomic_energies(atoms_h, W1, b1, W2, b2).reshape(n)
    acc = _segment_sum(energies, ids)
    return acc[:N_MOL]


# speedup vs baseline: 1.9314x; 1.0003x over previous
import functools

import jax
import jax.numpy as jnp
from jax import lax
from jax.experimental import pallas as pl
from jax.experimental.pallas import tpu as pltpu
from jax.experimental.pallas import tpu_sc as plsc

HIDDEN = 128
N_MOL = 1024
N_SUB = 16
CHUNK = 6400
N_PAD = N_SUB * CHUNK
BLK = 2000
ACC = N_MOL
COLS = ACC // N_SUB


def _mlp_body(x_ref, w1_ref, b1_ref, w2_ref, b2_ref, o_ref):
    x = x_ref[...]
    h = jnp.dot(x, w1_ref[...], preferred_element_type=jnp.float32)
    h = h + b1_ref[...]
    h = h * jax.nn.sigmoid(h)
    e = jnp.dot(h, w2_ref[...], preferred_element_type=jnp.float32)
    o_ref[...] = (e + b2_ref[...]).reshape(1, BLK // 16, 16)


def _atomic_energies(x, W1, b1, W2, b2):
    n = x.shape[0]
    grid = (n // BLK,)
    return pl.pallas_call(
        _mlp_body,
        grid=grid,
        in_specs=[
            pl.BlockSpec((BLK, HIDDEN), lambda i: (i, 0)),
            pl.BlockSpec((HIDDEN, HIDDEN), lambda i: (0, 0)),
            pl.BlockSpec((1, HIDDEN), lambda i: (0, 0)),
            pl.BlockSpec((HIDDEN, 1), lambda i: (0, 0)),
            pl.BlockSpec((1, 1), lambda i: (0, 0)),
        ],
        out_specs=pl.BlockSpec((1, BLK // 16, 16), lambda i: (i, 0, 0)),
        out_shape=jax.ShapeDtypeStruct((n // BLK, BLK // 16, 16), jnp.float32),
        compiler_params=pltpu.CompilerParams(
            dimension_semantics=("arbitrary",),
        ),
    )(x, W1, b1.reshape(1, HIDDEN), W2, b2.reshape(1, 1))


def _seg_body(n, e_hbm, id_hbm, out_hbm, e_v, id_v, acc_v, shr, tmp_v, res_v):
    last = n - (N_SUB - 1) * CHUNK
    w = lax.axis_index("s")
    base = w * CHUNK

    @pl.when(w < N_SUB - 1)
    def _():
        pltpu.sync_copy(e_hbm.at[pl.ds(base, CHUNK)], e_v)
        pltpu.sync_copy(id_hbm.at[pl.ds(base, CHUNK)], id_v)

    @pl.when(w == N_SUB - 1)
    def _():
        pltpu.sync_copy(e_hbm.at[pl.ds(base, last)], e_v.at[pl.ds(0, last)])
        pltpu.sync_copy(id_hbm.at[pl.ds(base, last)], id_v.at[pl.ds(0, last)])

    zeros = jnp.zeros((16,), jnp.float32)

    def zero_body(j, carry):
        acc_v[pl.ds(j * 16, 16)] = zeros
        return carry

    lax.fori_loop(0, ACC // 16, zero_body, 0, unroll=8)

    def scat_body(j, carry):
        idx = id_v[pl.ds(j * 16, 16)]
        val = e_v[pl.ds(j * 16, 16)]
        plsc.addupdate_scatter(acc_v, [idx], val)
        return carry

    lax.fori_loop(0, last // 16, scat_body, 0, unroll=8)

    @pl.when(w < N_SUB - 1)
    def _():
        lax.fori_loop(last // 16, CHUNK // 16, scat_body, 0, unroll=8)

    pltpu.sync_copy(acc_v, shr.at[pl.ds(w * ACC, ACC)])
    plsc.subcore_barrier()
    for r in range(N_SUB):
        pltpu.sync_copy(
            shr.at[pl.ds(r * ACC + w * COLS, COLS)],
            tmp_v.at[pl.ds(r * COLS, COLS)],
        )

    for c in range(COLS // 16):
        s = jnp.zeros((16,), jnp.float32)
        for r in range(N_SUB):
            s = s + tmp_v[pl.ds(r * COLS + c * 16, 16)]
        res_v[pl.ds(c * 16, 16)] = s

    pltpu.sync_copy(res_v, out_hbm.at[pl.ds(w * COLS, COLS)])


def _segment_sum(energies, ids):
    n = energies.shape[0]
    mesh = plsc.VectorSubcoreMesh(
        core_axis_name="c", subcore_axis_name="s", num_cores=1
    )
    fn = pl.kernel(
        functools.partial(_seg_body, n),
        mesh=mesh,
        out_type=jax.ShapeDtypeStruct((ACC,), jnp.float32),
        scratch_types=[
            pltpu.VMEM((CHUNK,), jnp.float32),
            pltpu.VMEM((CHUNK,), jnp.int32),
            pltpu.VMEM((ACC,), jnp.float32),
            pltpu.VMEM_SHARED((N_SUB * ACC,), jnp.float32),
            pltpu.VMEM((N_SUB * COLS,), jnp.float32),
            pltpu.VMEM((COLS,), jnp.float32),
        ],
        compiler_params=pltpu.CompilerParams(needs_layout_passes=False),
    )
    return fn(energies, ids)


@jax.jit
def kernel(atoms_h, graph_batch, W1, b1, W2, b2):
    n = atoms_h.shape[0]
    ids = graph_batch.astype(jnp.int32)
    energies = _atomic_energies(atoms_h, W1, b1, W2, b2).reshape(n)
    return _segment_sum(energies, ids)
